# baseline (speedup 1.0000x reference)
import functools

import jax
import jax.numpy as jnp
from jax import lax
from jax.experimental import pallas as pl
from jax.experimental.pallas import tpu as pltpu
from jax.experimental.pallas import tpu_sc as plsc

_B, _C, _T, _H, _W = 8, 3, 256, 64, 64
_PIX = _H * _W
_F = _B * _T
_NBINS = 512
_NW = 32
_FPW = _F // _NW
_LOOKUP = 101
_OUT_DIM = 128


_TCH = 16


def _tc_bins(frames):
    def body(x_ref, o_ref):
        x = x_ref[0]
        r, g, b2 = x[0], x[1], x[2]
        bins = ((r & 0xE0) << 1) | ((g & 0xE0) >> 2) | (b2 >> 5)
        wide = jnp.concatenate([bins[:, :32, :], bins[:, 32:, :]], axis=2)
        o_ref[...] = wide.reshape(_TCH * 32, 128)

    return pl.pallas_call(
        body,
        grid=(_B, _T // _TCH),
        in_specs=[pl.BlockSpec((1, _C, _TCH, _H, _W),
                               lambda i, j: (i, 0, j, 0, 0))],
        out_specs=pl.BlockSpec((_TCH * 32, 128),
                               lambda i, j: (i * (_T // _TCH) + j, 0)),
        out_shape=jax.ShapeDtypeStruct((_F * 32, 128), jnp.int32),
    )(frames)


def _sc_histograms(bins_flat):
    mesh = plsc.VectorSubcoreMesh(core_axis_name="c", subcore_axis_name="s")

    @functools.partial(
        pl.kernel,
        mesh=mesh,
        out_type=jax.ShapeDtypeStruct((_F * _NBINS,), jnp.float32),
        scratch_types=[
            pltpu.VMEM((2 * _PIX,), jnp.int32),
            pltpu.VMEM((2 * _NBINS,), jnp.float32),
            pltpu.SemaphoreType.DMA,
            pltpu.SemaphoreType.DMA,
            pltpu.SemaphoreType.DMA,
            pltpu.SemaphoreType.DMA,
        ],
        compiler_params=pltpu.CompilerParams(needs_layout_passes=False),
    )
    def hist_kernel(bins_hbm, hist_hbm, bins_v, hist_v,
                    in_sem0, in_sem1, out_sem0, out_sem1):
        wid = lax.axis_index("s") * 2 + lax.axis_index("c")
        f0 = wid * _FPW
        zeros16 = jnp.zeros((16,), jnp.float32)
        ones16 = jnp.ones((16,), jnp.float32)

        def in_copy(f, base, sem):
            return pltpu.make_async_copy(
                bins_hbm.at[pl.ds(f * _PIX, _PIX)],
                bins_v.at[pl.ds(base, _PIX)],
                sem)

        def out_copy(f, hbase, sem):
            return pltpu.make_async_copy(
                hist_v.at[pl.ds(hbase, _NBINS)],
                hist_hbm.at[pl.ds(f * _NBINS, _NBINS)],
                sem)

        def do_frame(f, base, hbase, hsem, j):
            @pl.when(j > 0)
            def _():
                out_copy(f, hbase, hsem).wait()

            @plsc.parallel_loop(0, _NBINS // 16, unroll=4)
            def _(i):
                hist_v[pl.ds(hbase + i * 16, 16)] = zeros16

            @plsc.parallel_loop(0, _PIX, step=16, unroll=8)
            def _(i):
                bins = bins_v[pl.ds(base + i, 16)]
                plsc.addupdate_scatter(
                    hist_v.at[pl.ds(hbase, _NBINS)], [bins], ones16)

            out_copy(f, hbase, hsem).start()

        in_copy(f0, 0, in_sem0).start()

        def pair_body(j, carry):
            f = f0 + 2 * j
            in_copy(f + 1, _PIX, in_sem1).start()
            in_copy(f, 0, in_sem0).wait()
            do_frame(f, 0, 0, out_sem0, j)

            @pl.when(j < _FPW // 2 - 1)
            def _():
                in_copy(f + 2, 0, in_sem0).start()

            in_copy(f + 1, _PIX, in_sem1).wait()
            do_frame(f + 1, _PIX, _NBINS, out_sem1, j)
            return carry

        lax.fori_loop(0, _FPW // 2, pair_body, 0)
        out_copy(f0 + _FPW - 2, 0, out_sem0).wait()
        out_copy(f0 + _FPW - 1, _NBINS, out_sem1).wait()

    return hist_kernel(bins_flat)


def _tc_post(hist3, w_pad, bias2):
    pad = (_LOOKUP - 1) // 2
    width = 512

    def body(x_ref, w_ref, b_ref, o_ref, sp_ref):
        x = x_ref[0]
        inv = 1.0 / jnp.sqrt(jnp.sum(x * x, axis=1, keepdims=True))
        xn = x * inv
        sims = lax.dot_general(xn, xn, (((1,), (1,)), ((), ())),
                               preferred_element_type=jnp.float32)
        sp_ref[...] = jnp.zeros((_T, width), jnp.float32)
        sp_ref[:, pad:pad + _T] = sims
        v = sp_ref[...]
        rows = lax.broadcasted_iota(jnp.int32, (_T, width), 0)
        for k in range(8):
            amt = 1 << k
            rolled = jnp.concatenate([v[:, amt:], v[:, :amt]], axis=1)
            v = jnp.where((rows & amt) != 0, rolled, v)
        band = v[:, :_OUT_DIM]
        out = lax.dot_general(band, w_ref[...], (((1,), (0,)), ((), ())),
                              preferred_element_type=jnp.float32)
        o_ref[0] = jnp.maximum(out + b_ref[...], 0.0)

    return pl.pallas_call(
        body,
        grid=(_B,),
        in_specs=[
            pl.BlockSpec((1, _T, _NBINS), lambda i: (i, 0, 0)),
            pl.BlockSpec((_OUT_DIM, _OUT_DIM), lambda i: (0, 0)),
            pl.BlockSpec((1, _OUT_DIM), lambda i: (0, 0)),
        ],
        out_specs=pl.BlockSpec((1, _T, _OUT_DIM), lambda i: (i, 0, 0)),
        out_shape=jax.ShapeDtypeStruct((_B, _T, _OUT_DIM), jnp.float32),
        scratch_shapes=[pltpu.VMEM((_T, width), jnp.float32)],
    )(hist3, w_pad, bias2)


def kernel(frames, Wfc, bfc):
    bins = _tc_bins(frames)
    hist = _sc_histograms(bins.reshape(_F * _PIX))
    hist3 = hist.reshape(_B, _T, _NBINS)
    w_pad = jnp.pad(Wfc.T, ((0, _OUT_DIM - _LOOKUP), (0, 0)))
    bias2 = bfc.reshape(1, _OUT_DIM)
    return _tc_post(hist3, w_pad, bias2)


# speedup vs baseline: 2.2176x; 2.2176x over previous
import functools

import jax
import jax.numpy as jnp
from jax import lax
from jax.experimental import pallas as pl
from jax.experimental.pallas import tpu as pltpu
from jax.experimental.pallas import tpu_sc as plsc

_B, _C, _T, _H, _W = 8, 3, 256, 64, 64
_PIX = _H * _W
_F = _B * _T
_NBINS = 512
_NW = 32
_FPW = _F // _NW
_LOOKUP = 101
_OUT_DIM = 128


_NTILES = _B * _C * _H * 16
_UBINS = 128 * _NBINS


def _sc_histograms(frames_tiles):
    mesh = plsc.VectorSubcoreMesh(core_axis_name="c", subcore_axis_name="s")

    _HROWS = _UBINS // 128

    @functools.partial(
        pl.kernel,
        mesh=mesh,
        out_type=jax.ShapeDtypeStruct((_F * 4, 128), jnp.float32),
        scratch_types=[
            pltpu.VMEM((6, 8, 128), jnp.int32),
            pltpu.VMEM((_HROWS, 128), jnp.float32),
            pltpu.VMEM((4, 128), jnp.int32),
            pltpu.VMEM_SHARED((8 * _HROWS, 128), jnp.float32),
            pltpu.SemaphoreType.DMA,
            pltpu.SemaphoreType.DMA,
        ],
        compiler_params=pltpu.CompilerParams(needs_layout_passes=False),
    )
    def hist_kernel(frames_hbm, hist_hbm, chunk_v, hist_v, idx_v, shared_v,
                    in_sem0, in_sem1):
        c_ax = lax.axis_index("c")
        s_ax = lax.axis_index("s")
        unit_local = s_ax // 2
        half = s_ax - 2 * unit_local
        ug = c_ax * 8 + unit_local
        b = ug // 2
        tg = ug - 2 * b
        base_n = b * (_C * 1024) + tg
        iota16 = lax.iota(jnp.int32, 16)
        iota4 = iota16 * 4
        ones16 = jnp.ones((16,), jnp.float32)
        zeros16 = jnp.zeros((16,), jnp.float32)

        def in_copies(i, slot, sem):
            m = 2 * i + half
            return [
                pltpu.make_async_copy(
                    frames_hbm.at[base_n + c * 1024 + m * 2],
                    chunk_v.at[slot * 3 + c],
                    sem)
                for c in range(_C)]

        def fire(i, slot, sem):
            for cp in in_copies(i, slot, sem):
                cp.start()

        def drain(i, slot, sem):
            for cp in in_copies(i, slot, sem):
                cp.wait()

        @plsc.parallel_loop(0, _UBINS // 16, unroll=8)
        def _(i):
            row = i // 8
            off = (i - row * 8) * 16
            hist_v[row, pl.ds(off, 16)] = zeros16

        for k in range(4):
            for i in range(8):
                idx_v[k, pl.ds(i * 16, 16)] = (
                    unit_local * _HROWS + k * 128 + i * 16 + iota16)

        def compute(slot):
            s3 = slot * 3

            @plsc.parallel_loop(0, 1024, step=16, unroll=8)
            def _(i):
                row = i // 128
                l = i - row * 128
                rv = chunk_v[s3, row, pl.ds(l, 16)]
                gv = chunk_v[s3 + 1, row, pl.ds(l, 16)]
                bv = chunk_v[s3 + 2, row, pl.ds(l, 16)]
                bins = ((rv & 0xE0) << 1) | ((gv & 0xE0) >> 2) | (bv >> 5)
                plsc.addupdate_scatter(
                    hist_v.at[pl.ds(l * 4, 64), :],
                    [iota4 + (bins >> 7), bins & 127], ones16)

        fire(0, 0, in_sem0)

        def pair_body(j, carry):
            i = 2 * j
            fire(i + 1, 1, in_sem1)
            drain(i, 0, in_sem0)
            compute(0)

            @pl.when(j < 127)
            def _():
                fire(i + 2, 0, in_sem0)

            drain(i + 1, 1, in_sem1)
            compute(1)
            return carry

        lax.fori_loop(0, 128, pair_body, 0)

        sh = shared_v.at[pl.ds(unit_local * _HROWS, _HROWS), :]

        @pl.when(half == 0)
        def _():
            pltpu.sync_copy(hist_v, sh)

        plsc.subcore_barrier()

        @pl.when(half == 1)
        def _():
            for k in range(4):
                pltpu.sync_copy(
                    hist_v.at[pl.ds(k * 128, 128), :],
                    shared_v.at[idx_v.at[k]],
                    add=True)

        plsc.subcore_barrier()

        @pl.when(half == 0)
        def _():
            pltpu.sync_copy(
                sh, hist_hbm.at[pl.ds((b * _T + tg * 128) * 4, _HROWS), :])

    return hist_kernel(frames_tiles)


def _tc_post(hist3, w_pad, bias2):
    pad = (_LOOKUP - 1) // 2
    width = 512

    def body(x_ref, w_ref, b_ref, o_ref, sp_ref):
        x = x_ref[0]
        inv = 1.0 / jnp.sqrt(jnp.sum(x * x, axis=1, keepdims=True))
        xn = x * inv
        sims = lax.dot_general(xn, xn, (((1,), (1,)), ((), ())),
                               preferred_element_type=jnp.float32)
        sp_ref[...] = jnp.zeros((_T, width), jnp.float32)
        sp_ref[:, pad:pad + _T] = sims
        v = sp_ref[...]
        rows = lax.broadcasted_iota(jnp.int32, (_T, width), 0)
        for k in range(8):
            amt = 1 << k
            rolled = jnp.concatenate([v[:, amt:], v[:, :amt]], axis=1)
            v = jnp.where((rows & amt) != 0, rolled, v)
        band = v[:, :_OUT_DIM]
        out = lax.dot_general(band, w_ref[...], (((1,), (0,)), ((), ())),
                              preferred_element_type=jnp.float32)
        o_ref[0] = jnp.maximum(out + b_ref[...], 0.0)

    return pl.pallas_call(
        body,
        grid=(_B,),
        in_specs=[
            pl.BlockSpec((1, _T, _NBINS), lambda i: (i, 0, 0)),
            pl.BlockSpec((_OUT_DIM, _OUT_DIM), lambda i: (0, 0)),
            pl.BlockSpec((1, _OUT_DIM), lambda i: (0, 0)),
        ],
        out_specs=pl.BlockSpec((1, _T, _OUT_DIM), lambda i: (i, 0, 0)),
        out_shape=jax.ShapeDtypeStruct((_B, _T, _OUT_DIM), jnp.float32),
        scratch_shapes=[pltpu.VMEM((_T, width), jnp.float32)],
    )(hist3, w_pad, bias2)


def kernel(frames, Wfc, bfc):
    frames_p = jnp.transpose(frames, (0, 1, 3, 4, 2))
    x7 = frames_p.reshape(_B, _C, _H, 8, 8, 2, 128)
    x7p = jnp.transpose(x7, (0, 1, 2, 3, 5, 4, 6))
    frames_tiles = x7p.reshape(_NTILES, 8, 128)
    hist = _sc_histograms(frames_tiles)
    hist3 = hist.reshape(_B, _T, _NBINS)
    w_pad = jnp.pad(Wfc.T, ((0, _OUT_DIM - _LOOKUP), (0, 0)))
    bias2 = bfc.reshape(1, _OUT_DIM)
    return _tc_post(hist3, w_pad, bias2)


# speedup vs baseline: 3.2774x; 1.4779x over previous
import functools

import jax
import jax.numpy as jnp
from jax import lax
from jax.experimental import pallas as pl
from jax.experimental.pallas import tpu as pltpu
from jax.experimental.pallas import tpu_sc as plsc

_B, _C, _T, _H, _W = 8, 3, 256, 64, 64
_PIX = _H * _W
_F = _B * _T
_NBINS = 512
_NW = 32
_FPW = _F // _NW
_LOOKUP = 101
_OUT_DIM = 128


_NTILES = _B * _C * _H * 16
_UBINS = 128 * _NBINS


def _sc_histograms(frames_tiles):
    mesh = plsc.VectorSubcoreMesh(core_axis_name="c", subcore_axis_name="s")

    _HROWS = _UBINS // 128

    @functools.partial(
        pl.kernel,
        mesh=mesh,
        out_type=jax.ShapeDtypeStruct((_F * 4, 128), jnp.float32),
        scratch_types=[
            pltpu.VMEM((12, 8, 128), jnp.int32),
            pltpu.VMEM((_HROWS, 128), jnp.float32),
            pltpu.VMEM((4, 128), jnp.int32),
            pltpu.VMEM_SHARED((8 * _HROWS, 128), jnp.float32),
            pltpu.SemaphoreType.DMA,
            pltpu.SemaphoreType.DMA,
            pltpu.SemaphoreType.DMA,
            pltpu.SemaphoreType.DMA,
        ],
        compiler_params=pltpu.CompilerParams(needs_layout_passes=False),
    )
    def hist_kernel(frames_hbm, hist_hbm, chunk_v, hist_v, idx_v, shared_v,
                    in_sem0, in_sem1, in_sem2, in_sem3):
        c_ax = lax.axis_index("c")
        s_ax = lax.axis_index("s")
        unit_local = s_ax // 2
        half = s_ax - 2 * unit_local
        ug = c_ax * 8 + unit_local
        b = ug // 2
        tg = ug - 2 * b
        base_n = b * (_C * 1024) + tg
        iota16 = lax.iota(jnp.int32, 16)
        iota4 = iota16 * 4
        ones16 = jnp.ones((16,), jnp.float32)
        zeros16 = jnp.zeros((16,), jnp.float32)

        def in_copies(i, slot, sem):
            m = 2 * i + half
            return [
                pltpu.make_async_copy(
                    frames_hbm.at[base_n + c * 1024 + m * 2],
                    chunk_v.at[slot * 3 + c],
                    sem)
                for c in range(_C)]

        def fire(i, slot, sem):
            for cp in in_copies(i, slot, sem):
                cp.start()

        def drain(i, slot, sem):
            for cp in in_copies(i, slot, sem):
                cp.wait()

        @plsc.parallel_loop(0, _UBINS // 16, unroll=8)
        def _(i):
            row = i // 8
            off = (i - row * 8) * 16
            hist_v[row, pl.ds(off, 16)] = zeros16

        for k in range(4):
            for i in range(8):
                idx_v[k, pl.ds(i * 16, 16)] = (
                    unit_local * _HROWS + k * 128 + i * 16 + iota16)

        def compute(slot):
            s3 = slot * 3

            @plsc.parallel_loop(0, 1024, step=16, unroll=8)
            def _(i):
                row = i // 128
                l = i - row * 128
                rv = chunk_v[s3, row, pl.ds(l, 16)]
                gv = chunk_v[s3 + 1, row, pl.ds(l, 16)]
                bv = chunk_v[s3 + 2, row, pl.ds(l, 16)]
                bins = ((rv & 0xE0) << 1) | ((gv & 0xE0) >> 2) | (bv >> 5)
                plsc.addupdate_scatter(
                    hist_v.at[pl.ds(l * 4, 64), :],
                    [iota4 + (bins >> 7), bins & 127], ones16)

        sems = [in_sem0, in_sem1, in_sem2, in_sem3]
        fire(0, 0, in_sem0)
        fire(1, 1, in_sem1)
        fire(2, 2, in_sem2)

        def quad_body(j, carry):
            i = 4 * j
            fire(i + 3, 3, in_sem3)
            for q in range(4):
                if q > 0:
                    @pl.when(j < 63)
                    def _(q=q):
                        fire(i + 3 + q, q - 1, sems[q - 1])
                drain(i + q, q, sems[q])
                compute(q)
            return carry

        lax.fori_loop(0, 64, quad_body, 0)

        sh = shared_v.at[pl.ds(unit_local * _HROWS, _HROWS), :]

        @pl.when(half == 0)
        def _():
            pltpu.sync_copy(hist_v, sh)

        plsc.subcore_barrier()

        @pl.when(half == 1)
        def _():
            for k in range(4):
                pltpu.sync_copy(
                    hist_v.at[pl.ds(k * 128, 128), :],
                    shared_v.at[idx_v.at[k]],
                    add=True)

        plsc.subcore_barrier()

        @pl.when(half == 0)
        def _():
            pltpu.sync_copy(
                sh, hist_hbm.at[pl.ds((b * _T + tg * 128) * 4, _HROWS), :])

    return hist_kernel(frames_tiles)


def _tc_post(hist3, w_pad, bias2):
    pad = (_LOOKUP - 1) // 2
    width = 512

    def body(x_ref, w_ref, b_ref, o_ref, sp_ref):
        x = x_ref[0]
        inv = 1.0 / jnp.sqrt(jnp.sum(x * x, axis=1, keepdims=True))
        xn = x * inv
        sims = lax.dot_general(xn, xn, (((1,), (1,)), ((), ())),
                               preferred_element_type=jnp.float32)
        sp_ref[...] = jnp.zeros((_T, width), jnp.float32)
        sp_ref[:, pad:pad + _T] = sims
        v = sp_ref[...]
        rows = lax.broadcasted_iota(jnp.int32, (_T, width), 0)
        for k in range(8):
            amt = 1 << k
            rolled = jnp.concatenate([v[:, amt:], v[:, :amt]], axis=1)
            v = jnp.where((rows & amt) != 0, rolled, v)
        band = v[:, :_OUT_DIM]
        out = lax.dot_general(band, w_ref[...], (((1,), (0,)), ((), ())),
                              preferred_element_type=jnp.float32)
        o_ref[0] = jnp.maximum(out + b_ref[...], 0.0)

    return pl.pallas_call(
        body,
        grid=(_B,),
        in_specs=[
            pl.BlockSpec((1, _T, _NBINS), lambda i: (i, 0, 0)),
            pl.BlockSpec((_OUT_DIM, _OUT_DIM), lambda i: (0, 0)),
            pl.BlockSpec((1, _OUT_DIM), lambda i: (0, 0)),
        ],
        out_specs=pl.BlockSpec((1, _T, _OUT_DIM), lambda i: (i, 0, 0)),
        out_shape=jax.ShapeDtypeStruct((_B, _T, _OUT_DIM), jnp.float32),
        scratch_shapes=[pltpu.VMEM((_T, width), jnp.float32)],
    )(hist3, w_pad, bias2)


def kernel(frames, Wfc, bfc):
    frames_p = jnp.transpose(frames, (0, 1, 3, 4, 2))
    x7 = frames_p.reshape(_B, _C, _H, 8, 8, 2, 128)
    x7p = jnp.transpose(x7, (0, 1, 2, 3, 5, 4, 6))
    frames_tiles = x7p.reshape(_NTILES, 8, 128)
    hist = _sc_histograms(frames_tiles)
    hist3 = hist.reshape(_B, _T, _NBINS)
    w_pad = jnp.pad(Wfc.T, ((0, _OUT_DIM - _LOOKUP), (0, 0)))
    bias2 = bfc.reshape(1, _OUT_DIM)
    return _tc_post(hist3, w_pad, bias2)


# speedup vs baseline: 3.5102x; 1.0710x over previous
import functools

import jax
import jax.numpy as jnp
from jax import lax
from jax.experimental import pallas as pl
from jax.experimental.pallas import tpu as pltpu
from jax.experimental.pallas import tpu_sc as plsc

_B, _C, _T, _H, _W = 8, 3, 256, 64, 64
_PIX = _H * _W
_F = _B * _T
_NBINS = 512
_NW = 32
_FPW = _F // _NW
_LOOKUP = 101
_OUT_DIM = 128


_NTILES = _B * _C * _H * 16
_UBINS = 128 * _NBINS


def _sc_histograms(frames_tiles):
    mesh = plsc.VectorSubcoreMesh(core_axis_name="c", subcore_axis_name="s")

    _HROWS = _UBINS // 128

    @functools.partial(
        pl.kernel,
        mesh=mesh,
        out_type=jax.ShapeDtypeStruct((_F * 4, 128), jnp.float32),
        scratch_types=[
            pltpu.VMEM((18, 8, 128), jnp.int32),
            pltpu.VMEM((_HROWS, 128), jnp.float32),
            pltpu.VMEM((4, 128), jnp.int32),
            pltpu.VMEM_SHARED((8 * _HROWS, 128), jnp.float32),
            pltpu.SemaphoreType.DMA,
            pltpu.SemaphoreType.DMA,
            pltpu.SemaphoreType.DMA,
            pltpu.SemaphoreType.DMA,
            pltpu.SemaphoreType.DMA,
            pltpu.SemaphoreType.DMA,
        ],
        compiler_params=pltpu.CompilerParams(needs_layout_passes=False),
    )
    def hist_kernel(frames_hbm, hist_hbm, chunk_v, hist_v, idx_v, shared_v,
                    in_sem0, in_sem1, in_sem2, in_sem3, in_sem4, in_sem5):
        c_ax = lax.axis_index("c")
        s_ax = lax.axis_index("s")
        unit_local = s_ax // 2
        half = s_ax - 2 * unit_local
        ug = c_ax * 8 + unit_local
        b = ug // 2
        tg = ug - 2 * b
        base_n = b * (_C * 1024) + tg
        iota16 = lax.iota(jnp.int32, 16)
        iota4 = iota16 * 4
        ones16 = jnp.ones((16,), jnp.float32)
        zeros16 = jnp.zeros((16,), jnp.float32)

        def in_copies(i, slot, sem):
            m = 2 * i + half
            return [
                pltpu.make_async_copy(
                    frames_hbm.at[base_n + c * 1024 + m * 2],
                    chunk_v.at[slot * 3 + c],
                    sem)
                for c in range(_C)]

        def fire(i, slot, sem):
            for cp in in_copies(i, slot, sem):
                cp.start()

        def drain(i, slot, sem):
            for cp in in_copies(i, slot, sem):
                cp.wait()

        @plsc.parallel_loop(0, _UBINS // 16, unroll=8)
        def _(i):
            row = i // 8
            off = (i - row * 8) * 16
            hist_v[row, pl.ds(off, 16)] = zeros16

        for k in range(4):
            for i in range(8):
                idx_v[k, pl.ds(i * 16, 16)] = (
                    unit_local * _HROWS + k * 128 + i * 16 + iota16)

        def compute(slot):
            s3 = slot * 3

            @plsc.parallel_loop(0, 1024, step=16, unroll=8)
            def _(i):
                row = i // 128
                l = i - row * 128
                rv = chunk_v[s3, row, pl.ds(l, 16)]
                gv = chunk_v[s3 + 1, row, pl.ds(l, 16)]
                bv = chunk_v[s3 + 2, row, pl.ds(l, 16)]
                bins = ((rv & 0xE0) << 1) | ((gv & 0xE0) >> 2) | (bv >> 5)
                plsc.addupdate_scatter(
                    hist_v.at[pl.ds(l * 4, 64), :],
                    [iota4 + (bins >> 7), bins & 127], ones16)

        sems = [in_sem0, in_sem1, in_sem2, in_sem3, in_sem4, in_sem5]
        _NS = 6
        _LA = 5
        for p in range(_LA):
            fire(p, p, sems[p])

        def ring_body(j, carry):
            i = _NS * j
            fire(i + _LA, _LA, sems[_LA])
            for q in range(_NS):
                if q > 0:
                    ahead = i + _LA + q
                    @pl.when(ahead < 256)
                    def _(q=q, ahead=ahead):
                        fire(ahead, q - 1, sems[q - 1])
                drain(i + q, q, sems[q])
                compute(q)
            return carry

        lax.fori_loop(0, 256 // _NS, ring_body, 0)
        tail0 = (256 // _NS) * _NS
        for r in range(tail0, 256):
            drain(r, r - tail0, sems[r - tail0])
            compute(r - tail0)

        sh = shared_v.at[pl.ds(unit_local * _HROWS, _HROWS), :]

        @pl.when(half == 0)
        def _():
            pltpu.sync_copy(hist_v, sh)

        plsc.subcore_barrier()

        @pl.when(half == 1)
        def _():
            for k in range(4):
                pltpu.sync_copy(
                    hist_v.at[pl.ds(k * 128, 128), :],
                    shared_v.at[idx_v.at[k]],
                    add=True)

        plsc.subcore_barrier()

        @pl.when(half == 0)
        def _():
            pltpu.sync_copy(
                sh, hist_hbm.at[pl.ds((b * _T + tg * 128) * 4, _HROWS), :])

    return hist_kernel(frames_tiles)


def _tc_post(hist3, w_pad, bias2):
    pad = (_LOOKUP - 1) // 2
    width = 512

    def body(x_ref, w_ref, b_ref, o_ref, sp_ref):
        x = x_ref[0]
        inv = 1.0 / jnp.sqrt(jnp.sum(x * x, axis=1, keepdims=True))
        xn = x * inv
        sims = lax.dot_general(xn, xn, (((1,), (1,)), ((), ())),
                               preferred_element_type=jnp.float32)
        sp_ref[...] = jnp.zeros((_T, width), jnp.float32)
        sp_ref[:, pad:pad + _T] = sims
        v = sp_ref[...]
        rows = lax.broadcasted_iota(jnp.int32, (_T, width), 0)
        for k in range(8):
            amt = 1 << k
            rolled = jnp.concatenate([v[:, amt:], v[:, :amt]], axis=1)
            v = jnp.where((rows & amt) != 0, rolled, v)
        band = v[:, :_OUT_DIM]
        out = lax.dot_general(band, w_ref[...], (((1,), (0,)), ((), ())),
                              preferred_element_type=jnp.float32)
        o_ref[0] = jnp.maximum(out + b_ref[...], 0.0)

    return pl.pallas_call(
        body,
        grid=(_B,),
        in_specs=[
            pl.BlockSpec((1, _T, _NBINS), lambda i: (i, 0, 0)),
            pl.BlockSpec((_OUT_DIM, _OUT_DIM), lambda i: (0, 0)),
            pl.BlockSpec((1, _OUT_DIM), lambda i: (0, 0)),
        ],
        out_specs=pl.BlockSpec((1, _T, _OUT_DIM), lambda i: (i, 0, 0)),
        out_shape=jax.ShapeDtypeStruct((_B, _T, _OUT_DIM), jnp.float32),
        scratch_shapes=[pltpu.VMEM((_T, width), jnp.float32)],
    )(hist3, w_pad, bias2)


def kernel(frames, Wfc, bfc):
    frames_p = jnp.transpose(frames, (0, 1, 3, 4, 2))
    x7 = frames_p.reshape(_B, _C, _H, 8, 8, 2, 128)
    x7p = jnp.transpose(x7, (0, 1, 2, 3, 5, 4, 6))
    frames_tiles = x7p.reshape(_NTILES, 8, 128)
    hist = _sc_histograms(frames_tiles)
    hist3 = hist.reshape(_B, _T, _NBINS)
    w_pad = jnp.pad(Wfc.T, ((0, _OUT_DIM - _LOOKUP), (0, 0)))
    bias2 = bfc.reshape(1, _OUT_DIM)
    return _tc_post(hist3, w_pad, bias2)


# speedup vs baseline: 3.5314x; 1.0060x over previous
import functools

import jax
import jax.numpy as jnp
from jax import lax
from jax.experimental import pallas as pl
from jax.experimental.pallas import tpu as pltpu
from jax.experimental.pallas import tpu_sc as plsc

_B, _C, _T, _H, _W = 8, 3, 256, 64, 64
_PIX = _H * _W
_F = _B * _T
_NBINS = 512
_NW = 32
_FPW = _F // _NW
_LOOKUP = 101
_OUT_DIM = 128


_NTILES = _B * _C * _H * 16
_UBINS = 128 * _NBINS


def _sc_histograms(frames_tiles):
    mesh = plsc.VectorSubcoreMesh(core_axis_name="c", subcore_axis_name="s")

    _HROWS = _UBINS // 128

    @functools.partial(
        pl.kernel,
        mesh=mesh,
        out_type=jax.ShapeDtypeStruct((_F * 4, 128), jnp.float32),
        scratch_types=[
            pltpu.VMEM((24, 8, 128), jnp.int32),
            pltpu.VMEM((_HROWS, 128), jnp.float32),
            pltpu.VMEM((4, 128), jnp.int32),
            pltpu.VMEM_SHARED((8 * _HROWS, 128), jnp.float32),
            pltpu.SemaphoreType.DMA,
            pltpu.SemaphoreType.DMA,
            pltpu.SemaphoreType.DMA,
            pltpu.SemaphoreType.DMA,
            pltpu.SemaphoreType.DMA,
            pltpu.SemaphoreType.DMA,
            pltpu.SemaphoreType.DMA,
            pltpu.SemaphoreType.DMA,
        ],
        compiler_params=pltpu.CompilerParams(needs_layout_passes=False),
    )
    def hist_kernel(frames_hbm, hist_hbm, chunk_v, hist_v, idx_v, shared_v,
                    in_sem0, in_sem1, in_sem2, in_sem3,
                    in_sem4, in_sem5, in_sem6, in_sem7):
        c_ax = lax.axis_index("c")
        s_ax = lax.axis_index("s")
        unit_local = s_ax // 2
        half = s_ax - 2 * unit_local
        ug = c_ax * 8 + unit_local
        b = ug // 2
        tg = ug - 2 * b
        base_n = b * (_C * 1024) + tg
        iota16 = lax.iota(jnp.int32, 16)
        iota4 = iota16 * 4
        ones16 = jnp.ones((16,), jnp.float32)
        zeros16 = jnp.zeros((16,), jnp.float32)

        def in_copies(i, slot, sem):
            m = 2 * i + half
            return [
                pltpu.make_async_copy(
                    frames_hbm.at[base_n + c * 1024 + m * 2],
                    chunk_v.at[slot * 3 + c],
                    sem)
                for c in range(_C)]

        def fire(i, slot, sem):
            for cp in in_copies(i, slot, sem):
                cp.start()

        def drain(i, slot, sem):
            for cp in in_copies(i, slot, sem):
                cp.wait()

        @plsc.parallel_loop(0, _UBINS // 16, unroll=8)
        def _(i):
            row = i // 8
            off = (i - row * 8) * 16
            hist_v[row, pl.ds(off, 16)] = zeros16

        for k in range(4):
            for i in range(8):
                idx_v[k, pl.ds(i * 16, 16)] = (
                    unit_local * _HROWS + k * 128 + i * 16 + iota16)

        def compute(slot):
            s3 = slot * 3

            @plsc.parallel_loop(0, 1024, step=16, unroll=8)
            def _(i):
                row = i // 128
                l = i - row * 128
                rv = chunk_v[s3, row, pl.ds(l, 16)]
                gv = chunk_v[s3 + 1, row, pl.ds(l, 16)]
                bv = chunk_v[s3 + 2, row, pl.ds(l, 16)]
                bins = ((rv & 0xE0) << 1) | ((gv & 0xE0) >> 2) | (bv >> 5)
                plsc.addupdate_scatter(
                    hist_v.at[pl.ds(l * 4, 64), :],
                    [iota4 + (bins >> 7), bins & 127], ones16)

        sems = [in_sem0, in_sem1, in_sem2, in_sem3,
                in_sem4, in_sem5, in_sem6, in_sem7]
        _NS = 8
        _LA = 7
        for p in range(_LA):
            fire(p, p, sems[p])

        def ring_body(j, carry):
            i = _NS * j
            fire(i + _LA, _LA, sems[_LA])
            for q in range(_NS):
                if q > 0:
                    ahead = i + _LA + q
                    @pl.when(ahead < 256)
                    def _(q=q, ahead=ahead):
                        fire(ahead, q - 1, sems[q - 1])
                drain(i + q, q, sems[q])
                compute(q)
            return carry

        lax.fori_loop(0, 256 // _NS, ring_body, 0)

        sh = shared_v.at[pl.ds(unit_local * _HROWS, _HROWS), :]

        @pl.when(half == 0)
        def _():
            pltpu.sync_copy(hist_v, sh)

        plsc.subcore_barrier()

        @pl.when(half == 1)
        def _():
            for k in range(4):
                pltpu.sync_copy(
                    hist_v.at[pl.ds(k * 128, 128), :],
                    shared_v.at[idx_v.at[k]],
                    add=True)

        plsc.subcore_barrier()

        @pl.when(half == 0)
        def _():
            pltpu.sync_copy(
                sh, hist_hbm.at[pl.ds((b * _T + tg * 128) * 4, _HROWS), :])

    return hist_kernel(frames_tiles)


def _tc_post(hist3, w_pad, bias2):
    pad = (_LOOKUP - 1) // 2
    width = 512

    def body(x_ref, w_ref, b_ref, o_ref, sp_ref):
        x = x_ref[0]
        inv = 1.0 / jnp.sqrt(jnp.sum(x * x, axis=1, keepdims=True))
        xn = x * inv
        sims = lax.dot_general(xn, xn, (((1,), (1,)), ((), ())),
                               preferred_element_type=jnp.float32)
        sp_ref[...] = jnp.zeros((_T, width), jnp.float32)
        sp_ref[:, pad:pad + _T] = sims
        v = sp_ref[...]
        rows = lax.broadcasted_iota(jnp.int32, (_T, width), 0)
        for k in range(8):
            amt = 1 << k
            rolled = jnp.concatenate([v[:, amt:], v[:, :amt]], axis=1)
            v = jnp.where((rows & amt) != 0, rolled, v)
        band = v[:, :_OUT_DIM]
        out = lax.dot_general(band, w_ref[...], (((1,), (0,)), ((), ())),
                              preferred_element_type=jnp.float32)
        o_ref[0] = jnp.maximum(out + b_ref[...], 0.0)

    return pl.pallas_call(
        body,
        grid=(_B,),
        in_specs=[
            pl.BlockSpec((1, _T, _NBINS), lambda i: (i, 0, 0)),
            pl.BlockSpec((_OUT_DIM, _OUT_DIM), lambda i: (0, 0)),
            pl.BlockSpec((1, _OUT_DIM), lambda i: (0, 0)),
        ],
        out_specs=pl.BlockSpec((1, _T, _OUT_DIM), lambda i: (i, 0, 0)),
        out_shape=jax.ShapeDtypeStruct((_B, _T, _OUT_DIM), jnp.float32),
        scratch_shapes=[pltpu.VMEM((_T, width), jnp.float32)],
    )(hist3, w_pad, bias2)


def kernel(frames, Wfc, bfc):
    frames_p = jnp.transpose(frames, (0, 1, 3, 4, 2))
    x7 = frames_p.reshape(_B, _C, _H, 8, 8, 2, 128)
    x7p = jnp.transpose(x7, (0, 1, 2, 3, 5, 4, 6))
    frames_tiles = x7p.reshape(_NTILES, 8, 128)
    hist = _sc_histograms(frames_tiles)
    hist3 = hist.reshape(_B, _T, _NBINS)
    w_pad = jnp.pad(Wfc.T, ((0, _OUT_DIM - _LOOKUP), (0, 0)))
    bias2 = bfc.reshape(1, _OUT_DIM)
    return _tc_post(hist3, w_pad, bias2)


# speedup vs baseline: 3.6587x; 1.0360x over previous
import functools

import jax
import jax.numpy as jnp
from jax import lax
from jax.experimental import pallas as pl
from jax.experimental.pallas import tpu as pltpu
from jax.experimental.pallas import tpu_sc as plsc

_B, _C, _T, _H, _W = 8, 3, 256, 64, 64
_PIX = _H * _W
_F = _B * _T
_NBINS = 512
_NW = 32
_FPW = _F // _NW
_LOOKUP = 101
_OUT_DIM = 128


_NTILES = _B * _C * _H * 16
_UBINS = 128 * _NBINS


def _sc_histograms(frames_tiles):
    mesh = plsc.VectorSubcoreMesh(core_axis_name="c", subcore_axis_name="s")

    _HROWS = _UBINS // 128

    @functools.partial(
        pl.kernel,
        mesh=mesh,
        out_type=jax.ShapeDtypeStruct((_F * 4, 128), jnp.float32),
        scratch_types=[
            pltpu.VMEM((24, 8, 128), jnp.int32),
            pltpu.VMEM((_HROWS, 128), jnp.float32),
            pltpu.VMEM((4, 128), jnp.int32),
            pltpu.VMEM_SHARED((8 * _HROWS, 128), jnp.float32),
            pltpu.SemaphoreType.DMA,
            pltpu.SemaphoreType.DMA,
            pltpu.SemaphoreType.DMA,
            pltpu.SemaphoreType.DMA,
            pltpu.SemaphoreType.DMA,
            pltpu.SemaphoreType.DMA,
            pltpu.SemaphoreType.DMA,
            pltpu.SemaphoreType.DMA,
        ],
        compiler_params=pltpu.CompilerParams(needs_layout_passes=False),
    )
    def hist_kernel(frames_hbm, hist_hbm, chunk_v, hist_v, idx_v, shared_v,
                    in_sem0, in_sem1, in_sem2, in_sem3,
                    in_sem4, in_sem5, in_sem6, in_sem7):
        c_ax = lax.axis_index("c")
        s_ax = lax.axis_index("s")
        unit_local = s_ax // 2
        half = s_ax - 2 * unit_local
        ug = c_ax * 8 + unit_local
        b = ug // 2
        tg = ug - 2 * b
        base_n = b * (_C * 1024) + tg
        iota16 = lax.iota(jnp.int32, 16)
        iota4 = iota16 * 4
        ones16 = jnp.ones((16,), jnp.float32)
        zeros16 = jnp.zeros((16,), jnp.float32)

        def in_copies(i, slot, sem):
            m = 2 * i + half
            return [
                pltpu.make_async_copy(
                    frames_hbm.at[base_n + c * 1024 + m * 2],
                    chunk_v.at[slot * 3 + c],
                    sem)
                for c in range(_C)]

        def fire(i, slot, sem):
            for cp in in_copies(i, slot, sem):
                cp.start()

        def drain(i, slot, sem):
            for cp in in_copies(i, slot, sem):
                cp.wait()

        @plsc.parallel_loop(0, _UBINS // 16, unroll=8)
        def _(i):
            row = i // 8
            off = (i - row * 8) * 16
            hist_v[row, pl.ds(off, 16)] = zeros16

        for k in range(4):
            for i in range(8):
                idx_v[k, pl.ds(i * 16, 16)] = (
                    unit_local * _HROWS + k * 128 + i * 16 + iota16)

        def compute(slot):
            s3 = slot * 3

            @plsc.parallel_loop(0, 1024, step=16, unroll=8)
            def _(i):
                row = i // 128
                l = i - row * 128
                rv = chunk_v[s3, row, pl.ds(l, 16)]
                gv = chunk_v[s3 + 1, row, pl.ds(l, 16)]
                bv = chunk_v[s3 + 2, row, pl.ds(l, 16)]
                bins = ((rv & 0xE0) << 1) | ((gv & 0xE0) >> 2) | (bv >> 5)
                plsc.addupdate_scatter(
                    hist_v.at[pl.ds(l * 4, 64), :],
                    [iota4 + (bins >> 7), bins & 127], ones16)

        sems = [in_sem0, in_sem1, in_sem2, in_sem3,
                in_sem4, in_sem5, in_sem6, in_sem7]
        _NS = 8
        _LA = 7
        for p in range(_LA):
            fire(p, p, sems[p])

        def ring_body(j, carry):
            i = _NS * j
            fire(i + _LA, _LA, sems[_LA])
            for q in range(_NS):
                if q > 0:
                    ahead = i + _LA + q
                    @pl.when(ahead < 256)
                    def _(q=q, ahead=ahead):
                        fire(ahead, q - 1, sems[q - 1])
                drain(i + q, q, sems[q])
                compute(q)
            return carry

        lax.fori_loop(0, 256 // _NS, ring_body, 0)

        sh = shared_v.at[pl.ds(unit_local * _HROWS, _HROWS), :]

        @pl.when(half == 0)
        def _():
            pltpu.sync_copy(hist_v, sh)

        plsc.subcore_barrier()

        @pl.when(half == 1)
        def _():
            for k in range(4):
                pltpu.sync_copy(
                    hist_v.at[pl.ds(k * 128, 128), :],
                    shared_v.at[idx_v.at[k]],
                    add=True)

        plsc.subcore_barrier()

        @pl.when(half == 0)
        def _():
            pltpu.sync_copy(
                sh, hist_hbm.at[pl.ds((b * _T + tg * 128) * 4, _HROWS), :])

    return hist_kernel(frames_tiles)


def _tc_post(hist4, w_pad, bias2):
    pad = (_LOOKUP - 1) // 2
    width = 384

    def body(x_ref, w_ref, b_ref, o_ref, sp_ref):
        x4 = x_ref[...].reshape(_T, 4, 128)
        x = jnp.concatenate([x4[:, k, :] for k in range(4)], axis=1)
        inv = 1.0 / jnp.sqrt(jnp.sum(x * x, axis=1, keepdims=True))
        xn = x * inv
        sims = lax.dot_general(xn, xn, (((1,), (1,)), ((), ())),
                               preferred_element_type=jnp.float32)
        sp_ref[...] = jnp.zeros((_T, width), jnp.float32)
        sp_ref[:, pad:pad + _T] = sims
        v = sp_ref[...]
        rows = lax.broadcasted_iota(jnp.int32, (_T, width), 0)
        for k in range(8):
            amt = 1 << k
            rolled = jnp.concatenate([v[:, amt:], v[:, :amt]], axis=1)
            v = jnp.where((rows & amt) != 0, rolled, v)
        band = v[:, :_OUT_DIM]
        out = lax.dot_general(band, w_ref[...], (((1,), (0,)), ((), ())),
                              preferred_element_type=jnp.float32)
        o_ref[0] = jnp.maximum(out + b_ref[...], 0.0)

    return pl.pallas_call(
        body,
        grid=(_B,),
        in_specs=[
            pl.BlockSpec((_T * 4, 128), lambda i: (i, 0)),
            pl.BlockSpec((_OUT_DIM, _OUT_DIM), lambda i: (0, 0)),
            pl.BlockSpec((1, _OUT_DIM), lambda i: (0, 0)),
        ],
        out_specs=pl.BlockSpec((1, _T, _OUT_DIM), lambda i: (i, 0, 0)),
        out_shape=jax.ShapeDtypeStruct((_B, _T, _OUT_DIM), jnp.float32),
        scratch_shapes=[pltpu.VMEM((_T, width), jnp.float32)],
    )(hist4, w_pad, bias2)


def kernel(frames, Wfc, bfc):
    frames_p = jnp.transpose(frames, (0, 1, 3, 4, 2))
    x7 = frames_p.reshape(_B, _C, _H, 8, 8, 2, 128)
    x7p = jnp.transpose(x7, (0, 1, 2, 3, 5, 4, 6))
    frames_tiles = x7p.reshape(_NTILES, 8, 128)
    hist = _sc_histograms(frames_tiles)
    w_pad = jnp.pad(Wfc.T, ((0, _OUT_DIM - _LOOKUP), (0, 0)))
    bias2 = bfc.reshape(1, _OUT_DIM)
    return _tc_post(hist, w_pad, bias2)


# speedup vs baseline: 3.8555x; 1.0538x over previous
import functools

import jax
import jax.numpy as jnp
from jax import lax
from jax.experimental import pallas as pl
from jax.experimental.pallas import tpu as pltpu
from jax.experimental.pallas import tpu_sc as plsc

_B, _C, _T, _H, _W = 8, 3, 256, 64, 64
_PIX = _H * _W
_F = _B * _T
_NBINS = 512
_NW = 32
_FPW = _F // _NW
_LOOKUP = 101
_OUT_DIM = 128


_NTILES = _B * _C * _H * 16
_UBINS = 128 * _NBINS


def _sc_histograms(frames_tiles):
    mesh = plsc.VectorSubcoreMesh(core_axis_name="c", subcore_axis_name="s")

    _HROWS = _UBINS // 128

    @functools.partial(
        pl.kernel,
        mesh=mesh,
        out_type=jax.ShapeDtypeStruct((2 * _F * 4, 128), jnp.float32),
        scratch_types=[
            pltpu.VMEM((24, 8, 128), jnp.int32),
            pltpu.VMEM((_HROWS, 128), jnp.float32),
            pltpu.SemaphoreType.DMA,
            pltpu.SemaphoreType.DMA,
            pltpu.SemaphoreType.DMA,
            pltpu.SemaphoreType.DMA,
            pltpu.SemaphoreType.DMA,
            pltpu.SemaphoreType.DMA,
            pltpu.SemaphoreType.DMA,
            pltpu.SemaphoreType.DMA,
        ],
        compiler_params=pltpu.CompilerParams(needs_layout_passes=False),
    )
    def hist_kernel(frames_hbm, hist_hbm, chunk_v, hist_v,
                    in_sem0, in_sem1, in_sem2, in_sem3,
                    in_sem4, in_sem5, in_sem6, in_sem7):
        c_ax = lax.axis_index("c")
        s_ax = lax.axis_index("s")
        unit_local = s_ax // 2
        half = s_ax - 2 * unit_local
        ug = c_ax * 8 + unit_local
        b = ug // 2
        tg = ug - 2 * b
        base_n = b * (_C * 1024) + tg
        iota16 = lax.iota(jnp.int32, 16)
        iota4 = iota16 * 4
        ones16 = jnp.ones((16,), jnp.float32)
        zeros16 = jnp.zeros((16,), jnp.float32)

        def in_copies(i, slot, sem):
            m = 2 * i + half
            return [
                pltpu.make_async_copy(
                    frames_hbm.at[base_n + c * 1024 + m * 2],
                    chunk_v.at[slot * 3 + c],
                    sem)
                for c in range(_C)]

        def fire(i, slot, sem):
            for cp in in_copies(i, slot, sem):
                cp.start()

        def drain(i, slot, sem):
            for cp in in_copies(i, slot, sem):
                cp.wait()

        @plsc.parallel_loop(0, _UBINS // 16, unroll=8)
        def _(i):
            row = i // 8
            off = (i - row * 8) * 16
            hist_v[row, pl.ds(off, 16)] = zeros16

        def compute(slot):
            s3 = slot * 3

            @plsc.parallel_loop(0, 1024, step=16, unroll=8)
            def _(i):
                row = i // 128
                l = i - row * 128
                rv = chunk_v[s3, row, pl.ds(l, 16)]
                gv = chunk_v[s3 + 1, row, pl.ds(l, 16)]
                bv = chunk_v[s3 + 2, row, pl.ds(l, 16)]
                bins = ((rv & 0xE0) << 1) | ((gv & 0xE0) >> 2) | (bv >> 5)
                plsc.addupdate_scatter(
                    hist_v.at[pl.ds(l * 4, 64), :],
                    [iota4 + (bins >> 7), bins & 127], ones16)

        sems = [in_sem0, in_sem1, in_sem2, in_sem3,
                in_sem4, in_sem5, in_sem6, in_sem7]
        _NS = 8
        _LA = 7
        for p in range(_LA):
            fire(p, p, sems[p])

        def ring_body(j, carry):
            i = _NS * j
            fire(i + _LA, _LA, sems[_LA])
            for q in range(_NS):
                if q > 0:
                    ahead = i + _LA + q
                    @pl.when(ahead < 256)
                    def _(q=q, ahead=ahead):
                        fire(ahead, q - 1, sems[q - 1])
                drain(i + q, q, sems[q])
                compute(q)
            return carry

        lax.fori_loop(0, 256 // _NS, ring_body, 0)

        pltpu.sync_copy(
            hist_v,
            hist_hbm.at[pl.ds((half * _F + b * _T + tg * 128) * 4, _HROWS), :])

    return hist_kernel(frames_tiles)


def _tc_post(hist4, w_pad, bias2):
    pad = (_LOOKUP - 1) // 2
    width = 384

    def body(x0_ref, x1_ref, w_ref, b_ref, o_ref, sp_ref):
        x4 = (x0_ref[...] + x1_ref[...]).reshape(_T, 4, 128)
        x = jnp.concatenate([x4[:, k, :] for k in range(4)], axis=1)
        inv = 1.0 / jnp.sqrt(jnp.sum(x * x, axis=1, keepdims=True))
        xn = x * inv
        sims = lax.dot_general(xn, xn, (((1,), (1,)), ((), ())),
                               preferred_element_type=jnp.float32)
        sp_ref[...] = jnp.zeros((_T, width), jnp.float32)
        sp_ref[:, pad:pad + _T] = sims
        v = sp_ref[...]
        rows = lax.broadcasted_iota(jnp.int32, (_T, width), 0)
        for k in range(8):
            amt = 1 << k
            rolled = jnp.concatenate([v[:, amt:], v[:, :amt]], axis=1)
            v = jnp.where((rows & amt) != 0, rolled, v)
        band = v[:, :_OUT_DIM]
        out = lax.dot_general(band, w_ref[...], (((1,), (0,)), ((), ())),
                              preferred_element_type=jnp.float32)
        o_ref[0] = jnp.maximum(out + b_ref[...], 0.0)

    return pl.pallas_call(
        body,
        grid=(_B,),
        in_specs=[
            pl.BlockSpec((_T * 4, 128), lambda i: (i, 0)),
            pl.BlockSpec((_T * 4, 128), lambda i: (_B + i, 0)),
            pl.BlockSpec((_OUT_DIM, _OUT_DIM), lambda i: (0, 0)),
            pl.BlockSpec((1, _OUT_DIM), lambda i: (0, 0)),
        ],
        out_specs=pl.BlockSpec((1, _T, _OUT_DIM), lambda i: (i, 0, 0)),
        out_shape=jax.ShapeDtypeStruct((_B, _T, _OUT_DIM), jnp.float32),
        scratch_shapes=[pltpu.VMEM((_T, width), jnp.float32)],
    )(hist4, hist4, w_pad, bias2)


def kernel(frames, Wfc, bfc):
    frames_p = jnp.transpose(frames, (0, 1, 3, 4, 2))
    x7 = frames_p.reshape(_B, _C, _H, 8, 8, 2, 128)
    x7p = jnp.transpose(x7, (0, 1, 2, 3, 5, 4, 6))
    frames_tiles = x7p.reshape(_NTILES, 8, 128)
    hist = _sc_histograms(frames_tiles)
    w_pad = jnp.pad(Wfc.T, ((0, _OUT_DIM - _LOOKUP), (0, 0)))
    bias2 = bfc.reshape(1, _OUT_DIM)
    return _tc_post(hist, w_pad, bias2)
